# baseline (speedup 1.0000x reference)
import functools

import jax
import jax.numpy as jnp
from jax import lax
from jax.experimental import pallas as pl
from jax.experimental.pallas import tpu as pltpu
from jax.experimental.pallas import tpu_sc as plsc

_K = 8
_EPS = 1e-10
_LANES = 128
_TC_ROWS = 32
_SC_ROWS = 32
_SC_V = 16
_NEG = -3.4e38


def _tc_block_kernel(scores_ref, g_ref, out_ref, *, n_cols):
    s = scores_ref[...] + g_ref[...]
    c = jnp.max(s, axis=1, keepdims=True)
    w = jnp.exp(s - c)
    kh = jnp.zeros_like(w)
    for t in range(_K):
        d = jnp.sum(w, axis=1, keepdims=True)
        oh = w * (1.0 / d)
        kh = kh + oh
        if t + 1 < _K:
            w = w * jnp.maximum(1.0 - oh, _EPS)

    rows = kh.shape[0]
    n_chunks = n_cols // _LANES

    neg = jnp.full((rows, _LANES), -jnp.inf, jnp.float32)
    accs = [neg] * _K
    for k in range(n_chunks):
        x = kh[:, k * _LANES:(k + 1) * _LANES]
        for j in range(_K):
            hi = jnp.maximum(accs[j], x)
            x = jnp.minimum(accs[j], x)
            accs[j] = hi

    kcum = jnp.zeros((rows, 1), jnp.float32)
    tval = jnp.full((rows, 1), -jnp.inf, jnp.float32)
    work = list(accs)
    for t in range(_K):
        m = work[0]
        for j in range(1, _K):
            m = jnp.maximum(m, work[j])
        v = jnp.max(m, axis=1, keepdims=True)
        eqs = (work[0] == v).astype(jnp.float32)
        for j in range(1, _K):
            eqs = eqs + (work[j] == v).astype(jnp.float32)
        cnt = jnp.sum(eqs, axis=1, keepdims=True)
        tval = jnp.where(kcum < 8.0, v, tval)
        kcum = kcum + cnt
        if t + 1 < _K:
            work = [jnp.where(wj == v, -jnp.inf, wj) for wj in work]

    ge = kh >= tval
    n_ge = jnp.sum(ge.astype(jnp.float32), axis=1, keepdims=True)
    exact = jnp.all(n_ge == 8.0)

    @pl.when(exact)
    def _():
        out_ref[...] = ge.astype(jnp.float32)

    @pl.when(jnp.logical_not(exact))
    def _():
        col = lax.broadcasted_iota(jnp.int32, kh.shape, 1)
        gt = kh > tval
        need = 8.0 - jnp.sum(gt.astype(jnp.float32), axis=1, keepdims=True)
        base = gt
        last = jnp.full((rows, 1), -1, jnp.int32)
        for t in range(_K):
            cand = jnp.where((kh == tval) & (col > last), col, n_cols)
            j = jnp.min(cand, axis=1, keepdims=True)
            take = (float(t) < need) & (j < n_cols)
            base = base | (take & (col == j))
            last = jnp.where(take, j, last)
        out_ref[...] = base.astype(jnp.float32)


def _tc_part(scores, g):
    b, n = scores.shape
    spec = pl.BlockSpec((_TC_ROWS, n), lambda i: (i, 0))
    return pl.pallas_call(
        functools.partial(_tc_block_kernel, n_cols=n),
        grid=(b // _TC_ROWS,),
        in_specs=[spec, spec],
        out_specs=spec,
        out_shape=jax.ShapeDtypeStruct((b, n), jnp.float32),
    )(scores, g)


def _sc_part(scores, g):
    b, n = scores.shape
    n_ch = n // _SC_V
    mesh = plsc.VectorSubcoreMesh(core_axis_name="c", subcore_axis_name="s")

    _dnums = lax.GatherDimensionNumbers(
        offset_dims=(), collapsed_slice_dims=(0,), start_index_map=(0,))

    def _shuf(x, perm):
        return lax.gather(x, perm[:, None], _dnums, (1,),
                          mode=lax.GatherScatterMode.PROMISE_IN_BOUNDS)

    def _lanes():
        return lax.broadcasted_iota(jnp.int32, (_SC_V,), 0)

    def _bfly(x, op):
        for sft in (8, 4, 2, 1):
            x = op(x, _shuf(x, _lanes() ^ sft))
        return x

    @functools.partial(
        pl.kernel, mesh=mesh,
        out_type=jax.ShapeDtypeStruct((b, n), jnp.float32),
        scratch_types=[
            pltpu.VMEM((n,), jnp.float32),
            pltpu.VMEM((n,), jnp.float32),
        ],
    )
    def sc_kernel(scores_hbm, g_hbm, out_hbm, wbuf, khbuf):
        wid = lax.axis_index("s") * 2 + lax.axis_index("c")
        pltpu.sync_copy(scores_hbm.at[wid], wbuf)
        pltpu.sync_copy(g_hbm.at[wid], khbuf)

        def ds(i):
            return pl.ds(i * _SC_V, _SC_V)

        def p1(i, macc):
            sv = wbuf[ds(i)] + khbuf[ds(i)]
            wbuf[ds(i)] = sv
            return jnp.maximum(macc, sv)

        macc = lax.fori_loop(0, n_ch, p1,
                             jnp.full((_SC_V,), _NEG, jnp.float32))
        c = _bfly(macc, jnp.maximum)

        def p2(i, dacc):
            x = jnp.exp(wbuf[ds(i)] - c)
            wbuf[ds(i)] = x
            return dacc + x

        dacc = lax.fori_loop(0, n_ch, p2, jnp.zeros((_SC_V,), jnp.float32))

        for t in range(_K - 1):
            rv = 1.0 / _bfly(dacc, jnp.add)

            def pit(i, dnext, t=t, rv=rv):
                wv = wbuf[ds(i)]
                oh = wv * rv
                khv = oh if t == 0 else khbuf[ds(i)] + oh
                khbuf[ds(i)] = khv
                wn = wv * jnp.maximum(1.0 - oh, _EPS)
                wbuf[ds(i)] = wn
                return dnext + wn

            dacc = lax.fori_loop(0, n_ch, pit,
                                 jnp.zeros((_SC_V,), jnp.float32))

        rv = 1.0 / _bfly(dacc, jnp.add)
        neg = jnp.full((_SC_V,), _NEG, jnp.float32)

        def p8(i, carry):
            accs = list(carry)
            khv = khbuf[ds(i)] + wbuf[ds(i)] * rv
            khbuf[ds(i)] = khv
            x = khv
            for j in range(_K):
                hi = jnp.maximum(accs[j], x)
                x = jnp.minimum(accs[j], x)
                accs[j] = hi
            return tuple(accs)

        accs = list(lax.fori_loop(0, n_ch, p8, (neg,) * _K))

        kcum = jnp.zeros((_SC_V,), jnp.float32)
        tval = jnp.zeros((_SC_V,), jnp.float32)
        for t in range(_K):
            m = accs[0]
            for j in range(1, _K):
                m = jnp.maximum(m, accs[j])
            v = _bfly(m, jnp.maximum)
            eqs = jnp.zeros((_SC_V,), jnp.float32)
            for j in range(_K):
                eqs = eqs + (1.0 - jnp.sign(v - accs[j]))
            cnt = _bfly(eqs, jnp.add)
            live = jnp.maximum(jnp.sign(8.0 - kcum), 0.0)
            tval = tval + live * (v - tval)
            kcum = kcum + cnt
            if t + 1 < _K:
                accs = [a + (1.0 - jnp.sign(v - a)) * (0.5 * _NEG - a)
                        for a in accs]

        def pth(i, ng):
            gef = 1.0 - jnp.maximum(-jnp.sign(khbuf[ds(i)] - tval), 0.0)
            wbuf[ds(i)] = gef
            return ng + gef

        ngv = lax.fori_loop(0, n_ch, pth, jnp.zeros((_SC_V,), jnp.float32))
        n_ge = _bfly(ngv, jnp.add)

        @pl.when(jnp.abs(n_ge[0] - 8.0) > 0.5)
        def _():
            def pgt(i, ngt):
                gtf = jnp.maximum(jnp.sign(khbuf[ds(i)] - tval), 0.0)
                return ngt + gtf

            ngtv = lax.fori_loop(0, n_ch, pgt,
                                 jnp.zeros((_SC_V,), jnp.float32))
            quota0 = 8.0 - _bfly(ngtv, jnp.add)

            def ptie(i, quota):
                khv = khbuf[ds(i)]
                sgn = jnp.sign(khv - tval)
                gtf = jnp.maximum(sgn, 0.0)
                eqf = 1.0 - jnp.abs(sgn)
                incl = eqf
                for sft in (1, 2, 4, 8):
                    shifted = _shuf(incl, jnp.maximum(_lanes() - sft, 0))
                    lmask = jnp.minimum(
                        _lanes().astype(jnp.float32) - (sft - 1), 1.0)
                    incl = incl + shifted * jnp.maximum(lmask, 0.0)
                excl = incl - eqf
                takef = eqf * jnp.maximum(jnp.sign(quota - excl), 0.0)
                wbuf[ds(i)] = jnp.minimum(gtf + takef, 1.0)
                return quota - _bfly(eqf, jnp.add)

            lax.fori_loop(0, n_ch, ptie, quota0)

        pltpu.sync_copy(wbuf, out_hbm.at[wid])

    return sc_kernel(scores, g)


def kernel(scores, g):
    b, n = scores.shape
    tc_rows = b - _SC_ROWS
    tc_out = _tc_part(scores[:tc_rows], g[:tc_rows])
    sc_out = _sc_part(scores[tc_rows:], g[tc_rows:])
    return jnp.concatenate([tc_out, sc_out], axis=0)


# speedup vs baseline: 3.3717x; 3.3717x over previous
import functools

import jax
import jax.numpy as jnp
from jax import lax
from jax.experimental import pallas as pl

_K = 8
_EPS = 1e-10
_LANES = 128
_ROWS = 32


def _block_kernel(scores_ref, g_ref, out_ref, *, n_cols):
    s = scores_ref[...] + g_ref[...]
    c = jnp.max(s, axis=1, keepdims=True)
    w = jnp.exp(s - c)
    kh = jnp.zeros_like(w)
    for t in range(_K):
        d = jnp.sum(w, axis=1, keepdims=True)
        oh = w * (1.0 / d)
        kh = kh + oh
        if t + 1 < _K:
            w = w * jnp.maximum(1.0 - oh, _EPS)

    rows = kh.shape[0]
    n_chunks = n_cols // _LANES

    neg = jnp.full((rows, _LANES), -jnp.inf, jnp.float32)
    accs = [neg] * _K
    for k in range(n_chunks):
        x = kh[:, k * _LANES:(k + 1) * _LANES]
        for j in range(_K):
            hi = jnp.maximum(accs[j], x)
            x = jnp.minimum(accs[j], x)
            accs[j] = hi

    kcum = jnp.zeros((rows, 1), jnp.float32)
    tval = jnp.full((rows, 1), -jnp.inf, jnp.float32)
    work = list(accs)
    for t in range(_K):
        m = work[0]
        for j in range(1, _K):
            m = jnp.maximum(m, work[j])
        v = jnp.max(m, axis=1, keepdims=True)
        eqs = (work[0] == v).astype(jnp.float32)
        for j in range(1, _K):
            eqs = eqs + (work[j] == v).astype(jnp.float32)
        cnt = jnp.sum(eqs, axis=1, keepdims=True)
        tval = jnp.where(kcum < 8.0, v, tval)
        kcum = kcum + cnt
        if t + 1 < _K:
            work = [jnp.where(wj == v, -jnp.inf, wj) for wj in work]

    ge = kh >= tval
    n_ge = jnp.sum(ge.astype(jnp.float32), axis=1, keepdims=True)
    exact = jnp.all(n_ge == 8.0)

    @pl.when(exact)
    def _():
        out_ref[...] = ge.astype(jnp.float32)

    @pl.when(jnp.logical_not(exact))
    def _():
        col = lax.broadcasted_iota(jnp.int32, kh.shape, 1)
        gt = kh > tval
        need = 8.0 - jnp.sum(gt.astype(jnp.float32), axis=1, keepdims=True)
        base = gt
        last = jnp.full((rows, 1), -1, jnp.int32)
        for t in range(_K):
            cand = jnp.where((kh == tval) & (col > last), col, n_cols)
            j = jnp.min(cand, axis=1, keepdims=True)
            take = (float(t) < need) & (j < n_cols)
            base = base | (take & (col == j))
            last = jnp.where(take, j, last)
        out_ref[...] = base.astype(jnp.float32)


def kernel(scores, g):
    b, n = scores.shape
    rows = _ROWS
    grid = (b // rows,)
    spec = pl.BlockSpec((rows, n), lambda i: (i, 0))
    return pl.pallas_call(
        functools.partial(_block_kernel, n_cols=n),
        grid=grid,
        in_specs=[spec, spec],
        out_specs=spec,
        out_shape=jax.ShapeDtypeStruct((b, n), jnp.float32),
    )(scores, g)
